# baseline (speedup 1.0000x reference)
import functools

import jax
import jax.numpy as jnp
from jax import lax
from jax.experimental import pallas as pl
from jax.experimental.pallas import tpu as pltpu
from jax.experimental.pallas import tpu_sc as plsc

N = 10000
E = 320000
D_IN = 128
H = 256
HH = H // 2

NC = 2
NS = 16
LANES = 16

NP = 10240
BR = 2048
ED = E // (NC * NS)
ES = E // NS
GT = 8
KC = 80
NR = NP // NS
RC = 128

_mesh = plsc.VectorSubcoreMesh(
    core_axis_name="c", subcore_axis_name="s", num_cores=NC, num_subcores=NS
)


@functools.partial(
    pl.kernel,
    out_type=jax.ShapeDtypeStruct((NC * NS, NP), jnp.float32),
    mesh=_mesh,
    scratch_types=[
        pltpu.VMEM((NP,), jnp.float32),
        pltpu.VMEM((2000,), jnp.int32),
    ],
    compiler_params=pltpu.CompilerParams(needs_layout_passes=False),
)
def _deg_kernel(dst_hbm, out_hbm, hist, dbuf):
    wid = lax.axis_index("s") * NC + lax.axis_index("c")

    def zero_body(i, _):
        hist[pl.ds(i * LANES, LANES)] = jnp.zeros((LANES,), jnp.float32)
        return _

    lax.fori_loop(0, NP // LANES, zero_body, None)

    ones = jnp.ones((LANES,), jnp.float32)
    ebase = wid * ED

    def outer(i, _):
        pltpu.sync_copy(dst_hbm.at[pl.ds(ebase + i * 2000, 2000)], dbuf)

        def inner(j, _):
            idx = dbuf[pl.ds(j * LANES, LANES)]
            plsc.addupdate_scatter(hist, [idx], ones)
            return _

        lax.fori_loop(0, 2000 // LANES, inner, None)
        return _

    lax.fori_loop(0, ED // 2000, outer, None)
    pltpu.sync_copy(hist, out_hbm.at[wid])


@functools.partial(
    pl.kernel,
    out_type=(
        jax.ShapeDtypeStruct((NP, HH), jnp.float32),
        jax.ShapeDtypeStruct((NP, HH), jnp.float32),
    ),
    mesh=_mesh,
    compiler_params=pltpu.CompilerParams(needs_layout_passes=False),
    scratch_types=[
        pltpu.VMEM_SHARED((NP, HH), jnp.float32),
        pltpu.VMEM((KC,), jnp.int32),
        pltpu.VMEM((KC,), jnp.int32),
        pltpu.VMEM((KC, HH), jnp.float32),
        pltpu.VMEM((RC, HH), jnp.float32),
        pltpu.SemaphoreType.DMA,
    ],
)
def _scatter_kernel(glo, ghi, src_hbm, dst_hbm, slo, shi,
                    acc, sidx, didx, rows, stage, sem):
    c = lax.axis_index("c")
    s = lax.axis_index("s")
    nbase = s * NR
    ebase = s * ES

    def run_half(g_hbm, out_hbm):
        def init_body(i, _):
            off = nbase + i * RC
            pltpu.sync_copy(g_hbm.at[pl.ds(off, RC)], stage)
            pltpu.sync_copy(stage, acc.at[pl.ds(off, RC)])
            return _

        lax.fori_loop(0, NR // RC, init_body, None)
        plsc.subcore_barrier()

        def edge_body(i, _):
            off = s * (E // GT) + i * KC
            pltpu.sync_copy(src_hbm.at[pl.ds(off, KC)], sidx)
            pltpu.sync_copy(dst_hbm.at[pl.ds(off, KC)], didx)
            pltpu.async_copy(g_hbm.at[sidx], rows, sem).wait()
            pltpu.sync_copy(rows, acc.at[didx], add=True)
            return _

        @pl.when(s < GT)
        def _():
            lax.fori_loop(0, (E // GT) // KC, edge_body, None)

        plsc.subcore_barrier()

        def drain_body(i, _):
            off = nbase + i * RC
            pltpu.sync_copy(acc.at[pl.ds(off, RC)], stage)
            pltpu.sync_copy(stage, out_hbm.at[pl.ds(off, RC)])
            return _

        lax.fori_loop(0, NR // RC, drain_body, None)

    @pl.when(c == 0)
    def _():
        run_half(glo, slo)

    @pl.when(c == 1)
    def _():
        run_half(ghi, shi)


def _dinv_block(degp_blk):
    deg = jnp.sum(degp_blk, axis=0) + 1.0
    return lax.rsqrt(deg)


def _tc1_body(x_ref, w_ref, degp_ref, glo_ref, ghi_ref):
    dinv = _dinv_block(degp_ref[...])
    h = jnp.dot(x_ref[...], w_ref[...], preferred_element_type=jnp.float32,
                precision=lax.Precision.HIGHEST)
    g = h * dinv[:, None]
    glo_ref[...] = g[:, :HH]
    ghi_ref[...] = g[:, HH:]


def _tc2_body(slo_ref, shi_ref, degp_ref, b_ref, w_ref, glo_ref, ghi_ref):
    dinv = _dinv_block(degp_ref[...])
    b = b_ref[...]
    alo = jax.nn.relu(slo_ref[...] * dinv[:, None] + b[:, :HH])
    ahi = jax.nn.relu(shi_ref[...] * dinv[:, None] + b[:, HH:])
    w = w_ref[...]
    h = (jnp.dot(alo, w[:HH, :], preferred_element_type=jnp.float32,
                 precision=lax.Precision.HIGHEST)
         + jnp.dot(ahi, w[HH:, :], preferred_element_type=jnp.float32,
                   precision=lax.Precision.HIGHEST))
    g = h * dinv[:, None]
    glo_ref[...] = g[:, :HH]
    ghi_ref[...] = g[:, HH:]


def _tc3_body(slo_ref, shi_ref, degp_ref, b2_ref, w_ref, b3_ref, out_ref):
    dinv = _dinv_block(degp_ref[...])
    b2 = b2_ref[...]
    alo = jax.nn.relu(slo_ref[...] * dinv[:, None] + b2[:, :HH])
    ahi = jax.nn.relu(shi_ref[...] * dinv[:, None] + b2[:, HH:])
    w = w_ref[...]
    out_ref[...] = (jnp.dot(alo, w[:HH, :], preferred_element_type=jnp.float32,
                            precision=lax.Precision.HIGHEST)
                    + jnp.dot(ahi, w[HH:, :], preferred_element_type=jnp.float32,
                              precision=lax.Precision.HIGHEST)
                    + b3_ref[...])


_GRID = (pl.cdiv(N, BR),)
_row_spec = lambda w: pl.BlockSpec((BR, w), lambda i: (i, 0))
_degp_spec = pl.BlockSpec((NC * NS, BR), lambda i: (0, i))
_full_spec = lambda a, b: pl.BlockSpec((a, b), lambda i: (0, 0))


def _tc1(x, W1, degp):
    return pl.pallas_call(
        _tc1_body,
        grid=_GRID,
        in_specs=[_row_spec(D_IN), _full_spec(D_IN, H), _degp_spec],
        out_specs=[_row_spec(HH), _row_spec(HH)],
        out_shape=[jax.ShapeDtypeStruct((NP, HH), jnp.float32)] * 2,
    )(x, W1, degp)


def _tc2(slo, shi, degp, b1, W2):
    return pl.pallas_call(
        _tc2_body,
        grid=_GRID,
        in_specs=[_row_spec(HH), _row_spec(HH), _degp_spec,
                  _full_spec(1, H), _full_spec(H, H)],
        out_specs=[_row_spec(HH), _row_spec(HH)],
        out_shape=[jax.ShapeDtypeStruct((NP, HH), jnp.float32)] * 2,
    )(slo, shi, degp, b1, W2)


def _tc3(slo, shi, degp, b2, W3, b3):
    return pl.pallas_call(
        _tc3_body,
        grid=_GRID,
        in_specs=[_row_spec(HH), _row_spec(HH), _degp_spec,
                  _full_spec(1, H), _full_spec(H, H), _full_spec(1, H)],
        out_specs=_row_spec(H),
        out_shape=jax.ShapeDtypeStruct((N, H), jnp.float32),
    )(slo, shi, degp, b2, W3, b3)


def kernel(x, edge_index, W1, b1, W2, b2, W3, b3):
    src = edge_index[0]
    dst = edge_index[1]
    b1r = b1.reshape(1, H)
    b2r = b2.reshape(1, H)
    b3r = b3.reshape(1, H)

    degp = _deg_kernel(dst)
    glo, ghi = _tc1(x, W1, degp)
    slo, shi = _scatter_kernel(glo, ghi, src, dst)
    glo2, ghi2 = _tc2(slo, shi, degp, b1r, W2)
    slo2, shi2 = _scatter_kernel(glo2, ghi2, src, dst)
    return _tc3(slo2, shi2, degp, b2r, W3, b3r)


# speedup vs baseline: 2.7447x; 2.7447x over previous
import functools

import jax
import jax.numpy as jnp
from jax import lax
from jax.experimental import pallas as pl
from jax.experimental.pallas import tpu as pltpu
from jax.experimental.pallas import tpu_sc as plsc

N = 10000
E = 320000
D_IN = 128
H = 256
HH = H // 2

NC = 2
NS = 16
LANES = 16

NP = 10240
BR = 2048
ED = E // (NC * NS)
ES = E // NS
KC = 80
NR = NP // NS
RC = 128

_mesh = plsc.VectorSubcoreMesh(
    core_axis_name="c", subcore_axis_name="s", num_cores=NC, num_subcores=NS
)


@functools.partial(
    pl.kernel,
    out_type=jax.ShapeDtypeStruct((NC * NS, NP), jnp.float32),
    mesh=_mesh,
    scratch_types=[
        pltpu.VMEM((NP,), jnp.float32),
        pltpu.VMEM((2000,), jnp.int32),
    ],
    compiler_params=pltpu.CompilerParams(needs_layout_passes=False),
)
def _deg_kernel(dst_hbm, out_hbm, hist, dbuf):
    wid = lax.axis_index("s") * NC + lax.axis_index("c")

    def zero_body(i, _):
        hist[pl.ds(i * LANES, LANES)] = jnp.zeros((LANES,), jnp.float32)
        return _

    lax.fori_loop(0, NP // LANES, zero_body, None)

    ones = jnp.ones((LANES,), jnp.float32)
    ebase = wid * ED

    def outer(i, _):
        pltpu.sync_copy(dst_hbm.at[pl.ds(ebase + i * 2000, 2000)], dbuf)

        def inner(j, _):
            idx = dbuf[pl.ds(j * LANES, LANES)]
            plsc.addupdate_scatter(hist, [idx], ones)
            return _

        lax.fori_loop(0, 2000 // LANES, inner, None)
        return _

    lax.fori_loop(0, ED // 2000, outer, None)
    pltpu.sync_copy(hist, out_hbm.at[wid])


@functools.partial(
    pl.kernel,
    out_type=(
        jax.ShapeDtypeStruct((NP, HH), jnp.float32),
        jax.ShapeDtypeStruct((NP, HH), jnp.float32),
    ),
    mesh=_mesh,
    compiler_params=pltpu.CompilerParams(needs_layout_passes=False),
    scratch_types=[
        pltpu.VMEM_SHARED((NP, HH), jnp.float32),
        pltpu.VMEM((ES,), jnp.int32),
        pltpu.VMEM((KC,), jnp.int32),
        pltpu.VMEM((KC,), jnp.int32),
        pltpu.VMEM((KC, HH), jnp.float32),
        pltpu.VMEM((RC, HH), jnp.float32),
        pltpu.SemaphoreType.DMA,
        pltpu.SemaphoreType.DMA,
        pltpu.SemaphoreType.DMA,
    ],
)
def _scatter_kernel(glo, ghi, src_hbm, dst_hbm, slo, shi,
                    acc, sidx_all, d0, d1, rows, stage, gsem, ds0, ds1):
    didx = [d0, d1]
    dsem = [ds0, ds1]
    c = lax.axis_index("c")
    s = lax.axis_index("s")
    nbase = s * NR
    ebase = s * ES

    def run_half(g_hbm, out_hbm):
        def init_body(i, _):
            off = nbase + i * RC
            pltpu.sync_copy(g_hbm.at[pl.ds(off, RC)], stage)
            pltpu.sync_copy(stage, acc.at[pl.ds(off, RC)])
            return _

        lax.fori_loop(0, NR // RC, init_body, None)
        plsc.subcore_barrier()

        pltpu.sync_copy(src_hbm.at[pl.ds(ebase, ES)], sidx_all)

        def issue_didx(j, b):
            pltpu.async_copy(dst_hbm.at[pl.ds(ebase + j * KC, KC)],
                             didx[b], dsem[b])

        def do_chunk(i, b, issue_next):
            if issue_next:
                issue_didx(i + 1, 1 - b)
            pltpu.async_copy(
                g_hbm.at[sidx_all.at[pl.ds(i * KC, KC)]], rows, gsem
            ).wait()
            pltpu.make_async_copy(dst_hbm.at[pl.ds(ebase + i * KC, KC)],
                                  didx[b], dsem[b]).wait()
            pltpu.sync_copy(rows, acc.at[didx[b]], add=True)

        issue_didx(0, 0)
        NCH = ES // KC

        def group_body(gi, _):
            for b in range(2):
                do_chunk(gi * 2 + b, b, True)
            return _

        lax.fori_loop(0, NCH // 2 - 1, group_body, None)
        do_chunk(NCH - 2, 0, True)
        do_chunk(NCH - 1, 1, False)
        plsc.subcore_barrier()

        def drain_body(i, _):
            off = nbase + i * RC
            pltpu.sync_copy(acc.at[pl.ds(off, RC)], stage)
            pltpu.sync_copy(stage, out_hbm.at[pl.ds(off, RC)])
            return _

        lax.fori_loop(0, NR // RC, drain_body, None)

    @pl.when(c == 0)
    def _():
        run_half(glo, slo)

    @pl.when(c == 1)
    def _():
        run_half(ghi, shi)


def _dinv_block(degp_blk):
    deg = jnp.sum(degp_blk, axis=0) + 1.0
    return lax.rsqrt(deg)


def _tc1_body(x_ref, w_ref, degp_ref, glo_ref, ghi_ref):
    dinv = _dinv_block(degp_ref[...])
    h = jnp.dot(x_ref[...], w_ref[...], preferred_element_type=jnp.float32,
                precision=lax.Precision.HIGHEST)
    g = h * dinv[:, None]
    glo_ref[...] = g[:, :HH]
    ghi_ref[...] = g[:, HH:]


def _tc2_body(slo_ref, shi_ref, degp_ref, b_ref, w_ref, glo_ref, ghi_ref):
    dinv = _dinv_block(degp_ref[...])
    b = b_ref[...]
    alo = jax.nn.relu(slo_ref[...] * dinv[:, None] + b[:, :HH])
    ahi = jax.nn.relu(shi_ref[...] * dinv[:, None] + b[:, HH:])
    w = w_ref[...]
    h = (jnp.dot(alo, w[:HH, :], preferred_element_type=jnp.float32,
                 precision=lax.Precision.HIGHEST)
         + jnp.dot(ahi, w[HH:, :], preferred_element_type=jnp.float32,
                   precision=lax.Precision.HIGHEST))
    g = h * dinv[:, None]
    glo_ref[...] = g[:, :HH]
    ghi_ref[...] = g[:, HH:]


def _tc3_body(slo_ref, shi_ref, degp_ref, b2_ref, w_ref, b3_ref, out_ref):
    dinv = _dinv_block(degp_ref[...])
    b2 = b2_ref[...]
    alo = jax.nn.relu(slo_ref[...] * dinv[:, None] + b2[:, :HH])
    ahi = jax.nn.relu(shi_ref[...] * dinv[:, None] + b2[:, HH:])
    w = w_ref[...]
    out_ref[...] = (jnp.dot(alo, w[:HH, :], preferred_element_type=jnp.float32,
                            precision=lax.Precision.HIGHEST)
                    + jnp.dot(ahi, w[HH:, :], preferred_element_type=jnp.float32,
                              precision=lax.Precision.HIGHEST)
                    + b3_ref[...])


_GRID = (pl.cdiv(N, BR),)
_row_spec = lambda w: pl.BlockSpec((BR, w), lambda i: (i, 0))
_degp_spec = pl.BlockSpec((NC * NS, BR), lambda i: (0, i))
_full_spec = lambda a, b: pl.BlockSpec((a, b), lambda i: (0, 0))


def _tc1(x, W1, degp):
    return pl.pallas_call(
        _tc1_body,
        grid=_GRID,
        in_specs=[_row_spec(D_IN), _full_spec(D_IN, H), _degp_spec],
        out_specs=[_row_spec(HH), _row_spec(HH)],
        out_shape=[jax.ShapeDtypeStruct((NP, HH), jnp.float32)] * 2,
    )(x, W1, degp)


def _tc2(slo, shi, degp, b1, W2):
    return pl.pallas_call(
        _tc2_body,
        grid=_GRID,
        in_specs=[_row_spec(HH), _row_spec(HH), _degp_spec,
                  _full_spec(1, H), _full_spec(H, H)],
        out_specs=[_row_spec(HH), _row_spec(HH)],
        out_shape=[jax.ShapeDtypeStruct((NP, HH), jnp.float32)] * 2,
    )(slo, shi, degp, b1, W2)


def _tc3(slo, shi, degp, b2, W3, b3):
    return pl.pallas_call(
        _tc3_body,
        grid=_GRID,
        in_specs=[_row_spec(HH), _row_spec(HH), _degp_spec,
                  _full_spec(1, H), _full_spec(H, H), _full_spec(1, H)],
        out_specs=_row_spec(H),
        out_shape=jax.ShapeDtypeStruct((N, H), jnp.float32),
    )(slo, shi, degp, b2, W3, b3)


def kernel(x, edge_index, W1, b1, W2, b2, W3, b3):
    src = edge_index[0]
    dst = edge_index[1]
    b1r = b1.reshape(1, H)
    b2r = b2.reshape(1, H)
    b3r = b3.reshape(1, H)

    degp = _deg_kernel(dst)
    glo, ghi = _tc1(x, W1, degp)
    slo, shi = _scatter_kernel(glo, ghi, src, dst)
    glo2, ghi2 = _tc2(slo, shi, degp, b1r, W2)
    slo2, shi2 = _scatter_kernel(glo2, ghi2, src, dst)
    return _tc3(slo2, shi2, degp, b2r, W3, b3r)


# speedup vs baseline: 4.4892x; 1.6356x over previous
import functools

import jax
import jax.numpy as jnp
from jax import lax
from jax.experimental import pallas as pl
from jax.experimental.pallas import tpu as pltpu
from jax.experimental.pallas import tpu_sc as plsc

N = 10000
E = 320000
D_IN = 128
H = 256
HH = H // 2

NC = 2
NS = 16
LANES = 16

NP = 10240
BR = 2048
ED = E // (NC * NS)
ES = E // NS
KC = 80
NR = NP // NS
RC = KC

_mesh = plsc.VectorSubcoreMesh(
    core_axis_name="c", subcore_axis_name="s", num_cores=NC, num_subcores=NS
)


@functools.partial(
    pl.kernel,
    out_type=jax.ShapeDtypeStruct((NC * NS, NP), jnp.float32),
    mesh=_mesh,
    scratch_types=[
        pltpu.VMEM((NP,), jnp.float32),
        pltpu.VMEM((2000,), jnp.int32),
    ],
    compiler_params=pltpu.CompilerParams(needs_layout_passes=False),
)
def _deg_kernel(dst_hbm, out_hbm, hist, dbuf):
    wid = lax.axis_index("s") * NC + lax.axis_index("c")

    def zero_body(i, _):
        hist[pl.ds(i * LANES, LANES)] = jnp.zeros((LANES,), jnp.float32)
        return _

    lax.fori_loop(0, NP // LANES, zero_body, None)

    ones = jnp.ones((LANES,), jnp.float32)
    ebase = wid * ED

    def outer(i, _):
        pltpu.sync_copy(dst_hbm.at[pl.ds(ebase + i * 2000, 2000)], dbuf)

        def inner(j, _):
            idx = dbuf[pl.ds(j * LANES, LANES)]
            plsc.addupdate_scatter(hist, [idx], ones)
            return _

        lax.fori_loop(0, 2000 // LANES, inner, None)
        return _

    lax.fori_loop(0, ED // 2000, outer, None)
    pltpu.sync_copy(hist, out_hbm.at[wid])


@functools.partial(
    pl.kernel,
    out_type=(
        jax.ShapeDtypeStruct((NP, HH), jnp.float32),
        jax.ShapeDtypeStruct((NP, HH), jnp.float32),
    ),
    mesh=_mesh,
    compiler_params=pltpu.CompilerParams(needs_layout_passes=False),
    scratch_types=[
        pltpu.VMEM_SHARED((NP, HH), jnp.float32),
        pltpu.VMEM((ES,), jnp.int32),
        pltpu.VMEM((KC,), jnp.int32),
        pltpu.VMEM((KC,), jnp.int32),
        pltpu.VMEM((KC, HH), jnp.float32),
        pltpu.VMEM((KC, HH), jnp.float32),
        pltpu.SemaphoreType.DMA,
        pltpu.SemaphoreType.DMA,
        pltpu.SemaphoreType.DMA,
        pltpu.SemaphoreType.DMA,
    ],
)
def _scatter_kernel(glo, ghi, src_hbm, dst_hbm, slo, shi,
                    acc, sidx_all, d0, d1, r0, r1, gs0, gs1, ds0, ds1):
    didx = [d0, d1]
    rows = [r0, r1]
    gsem = [gs0, gs1]
    dsem = [ds0, ds1]
    stage = rows[0]
    c = lax.axis_index("c")
    s = lax.axis_index("s")
    nbase = s * NR
    ebase = s * ES

    def run_half(g_hbm, out_hbm):
        def init_body(i, _):
            off = nbase + i * RC
            pltpu.sync_copy(g_hbm.at[pl.ds(off, RC)], stage)
            pltpu.sync_copy(stage, acc.at[pl.ds(off, RC)])
            return _

        lax.fori_loop(0, NR // RC, init_body, None)
        plsc.subcore_barrier()

        pltpu.sync_copy(src_hbm.at[pl.ds(ebase, ES)], sidx_all)

        def issue(j, b):
            pltpu.async_copy(dst_hbm.at[pl.ds(ebase + j * KC, KC)],
                             didx[b], dsem[b])
            pltpu.async_copy(g_hbm.at[sidx_all.at[pl.ds(j * KC, KC)]],
                             rows[b], gsem[b])

        def do_chunk(i, b, issue_next):
            pltpu.make_async_copy(g_hbm.at[pl.ds(0, KC)],
                                  rows[b], gsem[b]).wait()
            pltpu.make_async_copy(dst_hbm.at[pl.ds(ebase + i * KC, KC)],
                                  didx[b], dsem[b]).wait()
            pltpu.sync_copy(rows[b], acc.at[didx[b]], add=True)
            if issue_next:
                issue(i + 2, b)

        NCH = ES // KC
        issue(0, 0)
        issue(1, 1)

        def group_body(gi, _):
            for b in range(2):
                do_chunk(gi * 2 + b, b, True)
            return _

        lax.fori_loop(0, NCH // 2 - 1, group_body, None)
        do_chunk(NCH - 2, 0, False)
        do_chunk(NCH - 1, 1, False)
        plsc.subcore_barrier()

        def drain_body(i, _):
            off = nbase + i * RC
            pltpu.sync_copy(acc.at[pl.ds(off, RC)], stage)
            pltpu.sync_copy(stage, out_hbm.at[pl.ds(off, RC)])
            return _

        lax.fori_loop(0, NR // RC, drain_body, None)

    @pl.when(c == 0)
    def _():
        run_half(glo, slo)

    @pl.when(c == 1)
    def _():
        run_half(ghi, shi)


def _dinv_block(degp_blk):
    deg = jnp.sum(degp_blk, axis=0) + 1.0
    return lax.rsqrt(deg)


def _tc1_body(x_ref, w_ref, degp_ref, glo_ref, ghi_ref):
    dinv = _dinv_block(degp_ref[...])
    h = jnp.dot(x_ref[...], w_ref[...], preferred_element_type=jnp.float32,
                precision=lax.Precision.HIGHEST)
    g = h * dinv[:, None]
    glo_ref[...] = g[:, :HH]
    ghi_ref[...] = g[:, HH:]


def _tc2_body(slo_ref, shi_ref, degp_ref, b_ref, w_ref, glo_ref, ghi_ref):
    dinv = _dinv_block(degp_ref[...])
    b = b_ref[...]
    alo = jax.nn.relu(slo_ref[...] * dinv[:, None] + b[:, :HH])
    ahi = jax.nn.relu(shi_ref[...] * dinv[:, None] + b[:, HH:])
    w = w_ref[...]
    h = (jnp.dot(alo, w[:HH, :], preferred_element_type=jnp.float32,
                 precision=lax.Precision.HIGHEST)
         + jnp.dot(ahi, w[HH:, :], preferred_element_type=jnp.float32,
                   precision=lax.Precision.HIGHEST))
    g = h * dinv[:, None]
    glo_ref[...] = g[:, :HH]
    ghi_ref[...] = g[:, HH:]


def _tc3_body(slo_ref, shi_ref, degp_ref, b2_ref, w_ref, b3_ref, out_ref):
    dinv = _dinv_block(degp_ref[...])
    b2 = b2_ref[...]
    alo = jax.nn.relu(slo_ref[...] * dinv[:, None] + b2[:, :HH])
    ahi = jax.nn.relu(shi_ref[...] * dinv[:, None] + b2[:, HH:])
    w = w_ref[...]
    out_ref[...] = (jnp.dot(alo, w[:HH, :], preferred_element_type=jnp.float32,
                            precision=lax.Precision.HIGHEST)
                    + jnp.dot(ahi, w[HH:, :], preferred_element_type=jnp.float32,
                              precision=lax.Precision.HIGHEST)
                    + b3_ref[...])


_GRID = (pl.cdiv(N, BR),)
_row_spec = lambda w: pl.BlockSpec((BR, w), lambda i: (i, 0))
_degp_spec = pl.BlockSpec((NC * NS, BR), lambda i: (0, i))
_full_spec = lambda a, b: pl.BlockSpec((a, b), lambda i: (0, 0))


def _tc1(x, W1, degp):
    return pl.pallas_call(
        _tc1_body,
        grid=_GRID,
        in_specs=[_row_spec(D_IN), _full_spec(D_IN, H), _degp_spec],
        out_specs=[_row_spec(HH), _row_spec(HH)],
        out_shape=[jax.ShapeDtypeStruct((NP, HH), jnp.float32)] * 2,
    )(x, W1, degp)


def _tc2(slo, shi, degp, b1, W2):
    return pl.pallas_call(
        _tc2_body,
        grid=_GRID,
        in_specs=[_row_spec(HH), _row_spec(HH), _degp_spec,
                  _full_spec(1, H), _full_spec(H, H)],
        out_specs=[_row_spec(HH), _row_spec(HH)],
        out_shape=[jax.ShapeDtypeStruct((NP, HH), jnp.float32)] * 2,
    )(slo, shi, degp, b1, W2)


def _tc3(slo, shi, degp, b2, W3, b3):
    return pl.pallas_call(
        _tc3_body,
        grid=_GRID,
        in_specs=[_row_spec(HH), _row_spec(HH), _degp_spec,
                  _full_spec(1, H), _full_spec(H, H), _full_spec(1, H)],
        out_specs=_row_spec(H),
        out_shape=jax.ShapeDtypeStruct((N, H), jnp.float32),
    )(slo, shi, degp, b2, W3, b3)


def kernel(x, edge_index, W1, b1, W2, b2, W3, b3):
    src = edge_index[0]
    dst = edge_index[1]
    b1r = b1.reshape(1, H)
    b2r = b2.reshape(1, H)
    b3r = b3.reshape(1, H)

    degp = _deg_kernel(dst)
    glo, ghi = _tc1(x, W1, degp)
    slo, shi = _scatter_kernel(glo, ghi, src, dst)
    glo2, ghi2 = _tc2(slo, shi, degp, b1r, W2)
    slo2, shi2 = _scatter_kernel(glo2, ghi2, src, dst)
    return _tc3(slo2, shi2, degp, b2r, W3, b3r)


# speedup vs baseline: 4.6991x; 1.0467x over previous
import functools

import jax
import jax.numpy as jnp
from jax import lax
from jax.experimental import pallas as pl
from jax.experimental.pallas import tpu as pltpu
from jax.experimental.pallas import tpu_sc as plsc

N = 10000
E = 320000
D_IN = 128
H = 256
HH = H // 2

NC = 2
NS = 16
LANES = 16

NP = 10240
BR = 2048
ED = E // (NC * NS)
ES = E // NS
KC = 80
NR = NP // NS
RC = KC

_mesh = plsc.VectorSubcoreMesh(
    core_axis_name="c", subcore_axis_name="s", num_cores=NC, num_subcores=NS
)


@functools.partial(
    pl.kernel,
    out_type=jax.ShapeDtypeStruct((NC * NS, NP), jnp.float32),
    mesh=_mesh,
    scratch_types=[
        pltpu.VMEM((NP,), jnp.float32),
        pltpu.VMEM((2000,), jnp.int32),
    ],
    compiler_params=pltpu.CompilerParams(needs_layout_passes=False),
)
def _deg_kernel(dst_hbm, out_hbm, hist, dbuf):
    wid = lax.axis_index("s") * NC + lax.axis_index("c")

    def zero_body(i, _):
        hist[pl.ds(i * LANES, LANES)] = jnp.zeros((LANES,), jnp.float32)
        return _

    lax.fori_loop(0, NP // LANES, zero_body, None)

    ones = jnp.ones((LANES,), jnp.float32)
    ebase = wid * ED

    def outer(i, _):
        pltpu.sync_copy(dst_hbm.at[pl.ds(ebase + i * 2000, 2000)], dbuf)

        def inner(j, _):
            idx = dbuf[pl.ds(j * LANES, LANES)]
            plsc.addupdate_scatter(hist, [idx], ones)
            return _

        lax.fori_loop(0, 2000 // LANES, inner, None)
        return _

    lax.fori_loop(0, ED // 2000, outer, None)
    pltpu.sync_copy(hist, out_hbm.at[wid])


@functools.partial(
    pl.kernel,
    out_type=(
        jax.ShapeDtypeStruct((NP, HH), jnp.float32),
        jax.ShapeDtypeStruct((NP, HH), jnp.float32),
    ),
    mesh=_mesh,
    compiler_params=pltpu.CompilerParams(needs_layout_passes=False),
    scratch_types=[
        pltpu.VMEM_SHARED((NP, HH), jnp.float32),
        pltpu.VMEM((ES,), jnp.int32),
        pltpu.VMEM((KC,), jnp.int32),
        pltpu.VMEM((KC,), jnp.int32),
        pltpu.VMEM((KC, HH), jnp.float32),
        pltpu.VMEM((KC, HH), jnp.float32),
        pltpu.SemaphoreType.DMA,
        pltpu.SemaphoreType.DMA,
        pltpu.SemaphoreType.DMA,
        pltpu.SemaphoreType.DMA,
    ],
)
def _scatter_kernel(glo, ghi, src_hbm, dst_hbm, slo, shi,
                    acc, sidx_all, d0, d1, r0, r1, gs0, gs1, ds0, ds1):
    didx = [d0, d1]
    rows = [r0, r1]
    gsem = [gs0, gs1]
    dsem = [ds0, ds1]
    stage = rows[0]
    c = lax.axis_index("c")
    s = lax.axis_index("s")
    nbase = s * NR
    ebase = s * ES

    def run_half(g_hbm, out_hbm):
        def init_body(i, _):
            off = nbase + i * RC
            pltpu.sync_copy(g_hbm.at[pl.ds(off, RC)], stage)
            pltpu.sync_copy(stage, acc.at[pl.ds(off, RC)])
            return _

        lax.fori_loop(0, NR // RC, init_body, None)
        plsc.subcore_barrier()

        pltpu.sync_copy(src_hbm.at[pl.ds(ebase, ES)], sidx_all)

        def issue(j, b):
            pltpu.async_copy(dst_hbm.at[pl.ds(ebase + j * KC, KC)],
                             didx[b], dsem[b])
            pltpu.async_copy(g_hbm.at[sidx_all.at[pl.ds(j * KC, KC)]],
                             rows[b], gsem[b])

        def do_chunk(i, b, issue_next):
            pltpu.make_async_copy(g_hbm.at[pl.ds(0, KC)],
                                  rows[b], gsem[b]).wait()
            pltpu.make_async_copy(dst_hbm.at[pl.ds(ebase + i * KC, KC)],
                                  didx[b], dsem[b]).wait()
            pltpu.sync_copy(rows[b], acc.at[didx[b]], add=True)
            if issue_next:
                issue(i + 2, b)

        NCH = ES // KC
        issue(0, 0)
        issue(1, 1)

        def group_body(gi, _):
            for b in range(2):
                do_chunk(gi * 2 + b, b, True)
            return _

        lax.fori_loop(0, NCH // 2 - 1, group_body, None)
        do_chunk(NCH - 2, 0, False)
        do_chunk(NCH - 1, 1, False)
        plsc.subcore_barrier()

        def drain_body(i, _):
            off = nbase + i * RC
            pltpu.sync_copy(acc.at[pl.ds(off, RC)], stage)
            pltpu.sync_copy(stage, out_hbm.at[pl.ds(off, RC)])
            return _

        lax.fori_loop(0, NR // RC, drain_body, None)

    @pl.when(c == 0)
    def _():
        run_half(glo, slo)

    @pl.when(c == 1)
    def _():
        run_half(ghi, shi)


def _dinv_block(degp_blk):
    deg = jnp.sum(degp_blk, axis=0) + 1.0
    return lax.rsqrt(deg)


def _tc1_body(x_ref, w_ref, degp_ref, glo_ref, ghi_ref):
    dinv = _dinv_block(degp_ref[...])
    h = jnp.dot(x_ref[...], w_ref[...], preferred_element_type=jnp.float32)
    g = h * dinv[:, None]
    glo_ref[...] = g[:, :HH]
    ghi_ref[...] = g[:, HH:]


def _tc2_body(slo_ref, shi_ref, degp_ref, b_ref, w_ref, glo_ref, ghi_ref):
    dinv = _dinv_block(degp_ref[...])
    b = b_ref[...]
    alo = jax.nn.relu(slo_ref[...] * dinv[:, None] + b[:, :HH])
    ahi = jax.nn.relu(shi_ref[...] * dinv[:, None] + b[:, HH:])
    w = w_ref[...]
    h = (jnp.dot(alo, w[:HH, :], preferred_element_type=jnp.float32)
         + jnp.dot(ahi, w[HH:, :], preferred_element_type=jnp.float32))
    g = h * dinv[:, None]
    glo_ref[...] = g[:, :HH]
    ghi_ref[...] = g[:, HH:]


def _tc3_body(slo_ref, shi_ref, degp_ref, b2_ref, w_ref, b3_ref, out_ref):
    dinv = _dinv_block(degp_ref[...])
    b2 = b2_ref[...]
    alo = jax.nn.relu(slo_ref[...] * dinv[:, None] + b2[:, :HH])
    ahi = jax.nn.relu(shi_ref[...] * dinv[:, None] + b2[:, HH:])
    w = w_ref[...]
    out_ref[...] = (jnp.dot(alo, w[:HH, :], preferred_element_type=jnp.float32)
                    + jnp.dot(ahi, w[HH:, :], preferred_element_type=jnp.float32)
                    + b3_ref[...])


_GRID = (pl.cdiv(N, BR),)
_row_spec = lambda w: pl.BlockSpec((BR, w), lambda i: (i, 0))
_degp_spec = pl.BlockSpec((NC * NS, BR), lambda i: (0, i))
_full_spec = lambda a, b: pl.BlockSpec((a, b), lambda i: (0, 0))


def _tc1(x, W1, degp):
    return pl.pallas_call(
        _tc1_body,
        grid=_GRID,
        in_specs=[_row_spec(D_IN), _full_spec(D_IN, H), _degp_spec],
        out_specs=[_row_spec(HH), _row_spec(HH)],
        out_shape=[jax.ShapeDtypeStruct((NP, HH), jnp.float32)] * 2,
    )(x, W1, degp)


def _tc2(slo, shi, degp, b1, W2):
    return pl.pallas_call(
        _tc2_body,
        grid=_GRID,
        in_specs=[_row_spec(HH), _row_spec(HH), _degp_spec,
                  _full_spec(1, H), _full_spec(H, H)],
        out_specs=[_row_spec(HH), _row_spec(HH)],
        out_shape=[jax.ShapeDtypeStruct((NP, HH), jnp.float32)] * 2,
    )(slo, shi, degp, b1, W2)


def _tc3(slo, shi, degp, b2, W3, b3):
    return pl.pallas_call(
        _tc3_body,
        grid=_GRID,
        in_specs=[_row_spec(HH), _row_spec(HH), _degp_spec,
                  _full_spec(1, H), _full_spec(H, H), _full_spec(1, H)],
        out_specs=_row_spec(H),
        out_shape=jax.ShapeDtypeStruct((N, H), jnp.float32),
    )(slo, shi, degp, b2, W3, b3)


def kernel(x, edge_index, W1, b1, W2, b2, W3, b3):
    src = edge_index[0]
    dst = edge_index[1]
    b1r = b1.reshape(1, H)
    b2r = b2.reshape(1, H)
    b3r = b3.reshape(1, H)

    degp = _deg_kernel(dst)
    glo, ghi = _tc1(x, W1, degp)
    slo, shi = _scatter_kernel(glo, ghi, src, dst)
    glo2, ghi2 = _tc2(slo, shi, degp, b1r, W2)
    slo2, shi2 = _scatter_kernel(glo2, ghi2, src, dst)
    return _tc3(slo2, shi2, degp, b2r, W3, b3r)
